# baseline (speedup 1.0000x reference)
import jax
import jax.numpy as jnp
from jax import lax
from jax.experimental import pallas as pl
from jax.experimental.pallas import tpu as pltpu
from jax.experimental.pallas import tpu_sc as plsc

N = 10000
E = 160000
D = 256
OUT = 128

EA = 96000
EBB = E - EA

_DOT_T = (((1,), (1,)), ((), ()))


EB = 16000


def _make_scores(start, num):
    nsteps = num // EB
    step0 = start // EB

    def _body(x_ref, wa_ref, syn_ref, scores_ref, m_ref, q_s, m_s):
        i = pl.program_id(0)

        @pl.when(i == 0)
        def _():
            q_s[...] = lax.dot_general(x_ref[0:1], wa_ref[...], _DOT_T,
                                       preferred_element_type=jnp.float32)

        s = lax.dot_general(q_s[...], syn_ref[...], _DOT_T,
                            preferred_element_type=jnp.float32)
        scores_ref[pl.ds(i * EB, EB)] = s[0]
        bm = jnp.max(s, axis=1, keepdims=True)

        @pl.when(i == 0)
        def _():
            m_s[...] = bm

        @pl.when(i > 0)
        def _():
            m_s[...] = jnp.maximum(m_s[...], bm)

        @pl.when(i == nsteps - 1)
        def _():
            m_ref[...] = jnp.broadcast_to(m_s[...], (1, 16))

    def _call(x, wa, syn):
        return pl.pallas_call(
            _body,
            grid=(nsteps,),
            in_specs=[
                pl.BlockSpec((8, D), lambda i: (0, 0)),
                pl.BlockSpec((D, D), lambda i: (0, 0)),
                pl.BlockSpec((EB, D), lambda i: (i + step0, 0)),
            ],
            out_specs=[
                pl.BlockSpec((num,), lambda i: (0,)),
                pl.BlockSpec((1, 16), lambda i: (0, 0)),
            ],
            out_shape=[
                jax.ShapeDtypeStruct((num,), jnp.float32),
                jax.ShapeDtypeStruct((1, 16), jnp.float32),
            ],
            scratch_shapes=[
                pltpu.VMEM((1, D), jnp.float32),
                pltpu.VMEM((1, 1), jnp.float32),
            ],
        )(x, wa, syn)

    return _call


_scores_a = _make_scores(0, EA)
_scores_b = _make_scores(EA, EBB)


NC = 2
NS = 16
NW = NC * NS


def _make_scatter(off, num):
    chunk = num // NW
    assert chunk % 8 == 0 and chunk * NW == num
    buf = -(-chunk // 16) * 16
    nvec = buf // 16

    def _body(scores_hbm, ei_hbm, m_hbm, z_hbm,
              pdeg_hbm, pu_hbm,
              sc_v, src_v, dst_v, m_v, acc_deg, acc_u):
        wid = lax.axis_index("s") * NC + lax.axis_index("c")
        base = wid * chunk

        if buf != chunk:
            sc_v[pl.ds(buf - 16, 16)] = jnp.full((16,), -1e30, jnp.float32)
            src_v[pl.ds(buf - 16, 16)] = jnp.zeros((16,), jnp.int32)
            dst_v[pl.ds(buf - 16, 16)] = jnp.zeros((16,), jnp.int32)

        pltpu.sync_copy(scores_hbm.at[pl.ds(base, chunk)],
                        sc_v.at[pl.ds(0, chunk)])
        pltpu.sync_copy(ei_hbm.at[pl.ds(off + base, chunk)],
                        src_v.at[pl.ds(0, chunk)])
        pltpu.sync_copy(ei_hbm.at[pl.ds(E + off + base, chunk)],
                        dst_v.at[pl.ds(0, chunk)])
        pltpu.sync_copy(m_hbm.at[0], m_v)
        pltpu.sync_copy(z_hbm, acc_deg)
        pltpu.sync_copy(z_hbm, acc_u)

        mv = m_v[...]

        def body(j, carry):
            o = j * 16
            ewu = jnp.exp(sc_v[pl.ds(o, 16)] - mv)
            dstv = dst_v[pl.ds(o, 16)]
            srcv = src_v[pl.ds(o, 16)]
            plsc.addupdate_scatter(acc_deg, [dstv], ewu)
            v0 = jnp.where(dstv == 0, ewu, jnp.zeros((16,), jnp.float32))
            plsc.addupdate_scatter(acc_u, [srcv], v0)
            return carry

        lax.fori_loop(0, nvec, body, 0)

        pltpu.sync_copy(acc_deg, pdeg_hbm.at[wid])
        pltpu.sync_copy(acc_u, pu_hbm.at[wid])

    return pl.kernel(
        _body,
        out_type=[
            jax.ShapeDtypeStruct((NW, N), jnp.float32),
            jax.ShapeDtypeStruct((NW, N), jnp.float32),
        ],
        mesh=plsc.VectorSubcoreMesh(core_axis_name="c", subcore_axis_name="s",
                                    num_cores=NC, num_subcores=NS),
        compiler_params=pltpu.CompilerParams(needs_layout_passes=False),
        scratch_types=[
            pltpu.VMEM((buf,), jnp.float32),
            pltpu.VMEM((buf,), jnp.int32),
            pltpu.VMEM((buf,), jnp.int32),
            pltpu.VMEM((16,), jnp.float32),
            pltpu.VMEM((N,), jnp.float32),
            pltpu.VMEM((N,), jnp.float32),
        ],
    )


_scatter_a = _make_scatter(0, EA)
_scatter_b = _make_scatter(EA, EBB)


def _final_body(pdega_ref, pua_ref, pdegb_ref, pub_ref, ma_ref, mb_ref,
                x_ref, wg_ref, bg_ref, subj_ref, obj_ref,
                wo_ref, bo_ref, out_ref):
    ma = ma_ref[:, :1]
    mb = mb_ref[:, :1]
    mm = jnp.maximum(ma, mb)
    sa = jnp.exp(ma - mm)
    sb = jnp.exp(mb - mm)
    deg_star = (sa * jnp.sum(pdega_ref[...], axis=0, keepdims=True)
                + sb * jnp.sum(pdegb_ref[...], axis=0, keepdims=True))
    u_star = (sa * jnp.sum(pua_ref[...], axis=0, keepdims=True)
              + sb * jnp.sum(pub_ref[...], axis=0, keepdims=True))
    z = jnp.sum(deg_star, axis=1, keepdims=True)
    deg = deg_star / z + 1.0
    diz = lax.rsqrt(deg)
    a = diz * (u_star / z)
    iota = lax.broadcasted_iota(jnp.int32, a.shape, 1)
    a = a + jnp.where(iota == 0, diz, 0.0)
    pre = jnp.dot(a, x_ref[...], preferred_element_type=jnp.float32)
    o0 = diz[:, :1] * lax.dot_general(pre, wg_ref[...], _DOT_T,
                                      preferred_element_type=jnp.float32)
    o0 = o0 + bg_ref[...]
    cat = jnp.concatenate([o0, subj_ref[...], obj_ref[...]], axis=1)
    logits = lax.dot_general(cat, wo_ref[...], _DOT_T,
                             preferred_element_type=jnp.float32) + bo_ref[...]
    ls = logits - jnp.max(logits, axis=1, keepdims=True)
    out_ref[...] = ls - jnp.log(jnp.sum(jnp.exp(ls), axis=1, keepdims=True))


def _final_call(pdega, pua, pdegb, pub, ma, mb, x, wg, bg, subj, obj, wo, bo):
    return pl.pallas_call(
        _final_body,
        out_shape=jax.ShapeDtypeStruct((1, OUT), jnp.float32),
    )(pdega, pua, pdegb, pub, ma, mb, x, wg, bg, subj, obj, wo, bo)


def kernel(encoder_outputs, syn_embeddeds, subj, obj, edge_index,
           W_attn, W_gcn, b_gcn, W_out, b_out):
    ei_flat = edge_index.reshape(2 * E)
    zeros_n = jnp.zeros((N,), jnp.float32)
    scores_a, ma = _scores_a(encoder_outputs, W_attn, syn_embeddeds)
    pdega, pua = _scatter_a(scores_a, ei_flat, ma, zeros_n)
    scores_b, mb = _scores_b(encoder_outputs, W_attn, syn_embeddeds)
    pdegb, pub = _scatter_b(scores_b, ei_flat, mb, zeros_n)
    out = _final_call(pdega, pua, pdegb, pub, ma, mb,
                      encoder_outputs, W_gcn,
                      b_gcn.reshape(1, D), subj.reshape(1, D),
                      obj.reshape(1, D), W_out, b_out.reshape(1, OUT))
    return out


# speedup vs baseline: 1.0703x; 1.0703x over previous
import jax
import jax.numpy as jnp
from jax import lax
from jax.experimental import pallas as pl
from jax.experimental.pallas import tpu as pltpu
from jax.experimental.pallas import tpu_sc as plsc

N = 10000
E = 160000
D = 256
OUT = 128

E2 = E // 2

_DOT_T = (((1,), (1,)), ((), ()))


EB = 16000
NSTEPS = E2 // EB
EIB = 2 * E // NSTEPS


def _make_scores(half):
    step0 = half * NSTEPS
    emit_ei = half == 0

    def _body(x_ref, wa_ref, syn_ref, *rest):
        if emit_ei:
            ei_ref, scores_ref, m_ref, ei_out, q_s, m_s = rest
        else:
            scores_ref, m_ref, q_s, m_s = rest
        i = pl.program_id(0)

        @pl.when(i == 0)
        def _():
            q_s[...] = lax.dot_general(x_ref[0:1], wa_ref[...], _DOT_T,
                                       preferred_element_type=jnp.float32)

        s = lax.dot_general(q_s[...], syn_ref[...], _DOT_T,
                            preferred_element_type=jnp.float32)
        scores_ref[pl.ds(i * EB, EB)] = s[0]
        bm = jnp.max(s, axis=1, keepdims=True)

        if emit_ei:
            half_b = EIB // 2
            ei_out[pl.ds(i * half_b, half_b)] = ei_ref[0]
            ei_out[pl.ds(E + i * half_b, half_b)] = ei_ref[1]

        @pl.when(i == 0)
        def _():
            m_s[...] = bm

        @pl.when(i > 0)
        def _():
            m_s[...] = jnp.maximum(m_s[...], bm)

        @pl.when(i == NSTEPS - 1)
        def _():
            m_ref[...] = jnp.broadcast_to(m_s[...], (1, 16))

    in_specs = [
        pl.BlockSpec((8, D), lambda i: (0, 0)),
        pl.BlockSpec((D, D), lambda i: (0, 0)),
        pl.BlockSpec((EB, D), lambda i: (i + step0, 0)),
    ]
    out_specs = [
        pl.BlockSpec((E2,), lambda i: (0,)),
        pl.BlockSpec((1, 16), lambda i: (0, 0)),
    ]
    out_shape = [
        jax.ShapeDtypeStruct((E2,), jnp.float32),
        jax.ShapeDtypeStruct((1, 16), jnp.float32),
    ]
    if emit_ei:
        in_specs.append(pl.BlockSpec((2, EIB // 2), lambda i: (0, i)))
        out_specs.append(pl.BlockSpec((2 * E,), lambda i: (0,)))
        out_shape.append(jax.ShapeDtypeStruct((2 * E,), jnp.int32))

    def _call(x, wa, syn, ei=None):
        args = (x, wa, syn) + ((ei,) if emit_ei else ())
        return pl.pallas_call(
            _body,
            grid=(NSTEPS,),
            in_specs=in_specs,
            out_specs=out_specs,
            out_shape=out_shape,
            scratch_shapes=[
                pltpu.VMEM((1, D), jnp.float32),
                pltpu.VMEM((1, 1), jnp.float32),
            ],
        )(*args)

    return _call


_scores_a = _make_scores(0)
_scores_b = _make_scores(1)


NC = 2
NS = 16
NW = NC * NS
CHUNK = (E2 // NW) // 8 * 8
EXTRA = E2 - NW * CHUNK
BUF = CHUNK + EXTRA
NVEC = BUF // 16


def _make_scatter(off):
    def _body(scores_hbm, ei_hbm, m_hbm,
              pdeg_hbm, pu_hbm,
              sc_v, src_v, dst_v, m_v, acc_deg, acc_u):
        wid = lax.axis_index("s") * NC + lax.axis_index("c")
        base = wid * CHUNK

        def fill_body(i, carry):
            sc_v[pl.ds(i * 16, 16)] = jnp.full((16,), -1e30, jnp.float32)
            src_v[pl.ds(i * 16, 16)] = jnp.zeros((16,), jnp.int32)
            dst_v[pl.ds(i * 16, 16)] = jnp.zeros((16,), jnp.int32)
            return carry

        lax.fori_loop(CHUNK // 16, NVEC, fill_body, 0)

        pltpu.sync_copy(scores_hbm.at[pl.ds(base, CHUNK)],
                        sc_v.at[pl.ds(0, CHUNK)])
        pltpu.sync_copy(ei_hbm.at[pl.ds(off + base, CHUNK)],
                        src_v.at[pl.ds(0, CHUNK)])
        pltpu.sync_copy(ei_hbm.at[pl.ds(E + off + base, CHUNK)],
                        dst_v.at[pl.ds(0, CHUNK)])
        pltpu.sync_copy(m_hbm.at[0], m_v)

        @pl.when(wid == 0)
        def _():
            tail = NW * CHUNK
            pltpu.sync_copy(scores_hbm.at[pl.ds(tail, EXTRA)],
                            sc_v.at[pl.ds(CHUNK, EXTRA)])
            pltpu.sync_copy(ei_hbm.at[pl.ds(off + tail, EXTRA)],
                            src_v.at[pl.ds(CHUNK, EXTRA)])
            pltpu.sync_copy(ei_hbm.at[pl.ds(E + off + tail, EXTRA)],
                            dst_v.at[pl.ds(CHUNK, EXTRA)])

        def zero_body(i, carry):
            z = jnp.zeros((16,), jnp.float32)
            acc_deg[pl.ds(i * 16, 16)] = z
            acc_u[pl.ds(i * 16, 16)] = z
            return carry

        lax.fori_loop(0, N // 16, zero_body, 0)

        mv = m_v[...]

        def body(j, carry):
            o = j * 16
            ewu = jnp.exp(sc_v[pl.ds(o, 16)] - mv)
            dstv = dst_v[pl.ds(o, 16)]
            srcv = src_v[pl.ds(o, 16)]
            plsc.addupdate_scatter(acc_deg, [dstv], ewu)
            v0 = jnp.where(dstv == 0, ewu, jnp.zeros((16,), jnp.float32))
            plsc.addupdate_scatter(acc_u, [srcv], v0)
            return carry

        lax.fori_loop(0, NVEC, body, 0)

        pltpu.sync_copy(acc_deg, pdeg_hbm.at[wid])
        pltpu.sync_copy(acc_u, pu_hbm.at[wid])

    return pl.kernel(
        _body,
        out_type=[
            jax.ShapeDtypeStruct((NW, N), jnp.float32),
            jax.ShapeDtypeStruct((NW, N), jnp.float32),
        ],
        mesh=plsc.VectorSubcoreMesh(core_axis_name="c", subcore_axis_name="s",
                                    num_cores=NC, num_subcores=NS),
        compiler_params=pltpu.CompilerParams(needs_layout_passes=False),
        scratch_types=[
            pltpu.VMEM((BUF,), jnp.float32),
            pltpu.VMEM((BUF,), jnp.int32),
            pltpu.VMEM((BUF,), jnp.int32),
            pltpu.VMEM((16,), jnp.float32),
            pltpu.VMEM((N,), jnp.float32),
            pltpu.VMEM((N,), jnp.float32),
        ],
    )


_scatter_a = _make_scatter(0)
_scatter_b = _make_scatter(E2)


def _final_body(pdega_ref, pua_ref, pdegb_ref, pub_ref, ma_ref, mb_ref,
                x_ref, wg_ref, bg_ref, subj_ref, obj_ref,
                wo_ref, bo_ref, out_ref):
    ma = ma_ref[:, :1]
    mb = mb_ref[:, :1]
    mm = jnp.maximum(ma, mb)
    sa = jnp.exp(ma - mm)
    sb = jnp.exp(mb - mm)
    deg_star = (sa * jnp.sum(pdega_ref[...], axis=0, keepdims=True)
                + sb * jnp.sum(pdegb_ref[...], axis=0, keepdims=True))
    u_star = (sa * jnp.sum(pua_ref[...], axis=0, keepdims=True)
              + sb * jnp.sum(pub_ref[...], axis=0, keepdims=True))
    z = jnp.sum(deg_star, axis=1, keepdims=True)
    deg = deg_star / z + 1.0
    diz = lax.rsqrt(deg)
    a = diz * (u_star / z)
    iota = lax.broadcasted_iota(jnp.int32, a.shape, 1)
    a = a + jnp.where(iota == 0, diz, 0.0)
    pre = jnp.dot(a, x_ref[...], preferred_element_type=jnp.float32)
    o0 = diz[:, :1] * lax.dot_general(pre, wg_ref[...], _DOT_T,
                                      preferred_element_type=jnp.float32)
    o0 = o0 + bg_ref[...]
    cat = jnp.concatenate([o0, subj_ref[...], obj_ref[...]], axis=1)
    logits = lax.dot_general(cat, wo_ref[...], _DOT_T,
                             preferred_element_type=jnp.float32) + bo_ref[...]
    ls = logits - jnp.max(logits, axis=1, keepdims=True)
    out_ref[...] = ls - jnp.log(jnp.sum(jnp.exp(ls), axis=1, keepdims=True))


def _final_call(pdega, pua, pdegb, pub, ma, mb, x, wg, bg, subj, obj, wo, bo):
    return pl.pallas_call(
        _final_body,
        out_shape=jax.ShapeDtypeStruct((1, OUT), jnp.float32),
    )(pdega, pua, pdegb, pub, ma, mb, x, wg, bg, subj, obj, wo, bo)


def kernel(encoder_outputs, syn_embeddeds, subj, obj, edge_index,
           W_attn, W_gcn, b_gcn, W_out, b_out):
    scores_a, ma, ei_lin = _scores_a(encoder_outputs, W_attn, syn_embeddeds,
                                     edge_index)
    pdega, pua = _scatter_a(scores_a, ei_lin, ma)
    scores_b, mb = _scores_b(encoder_outputs, W_attn, syn_embeddeds)
    pdegb, pub = _scatter_b(scores_b, ei_lin, mb)
    out = _final_call(pdega, pua, pdegb, pub, ma, mb,
                      encoder_outputs, W_gcn,
                      b_gcn.reshape(1, D), subj.reshape(1, D),
                      obj.reshape(1, D), W_out, b_out.reshape(1, OUT))
    return out
